# baseline (speedup 1.0000x reference)
import functools

import jax
import jax.numpy as jnp
import numpy as np
from jax.experimental import pallas as pl
from jax.experimental.pallas import tpu as pltpu

NSAMP = 8192
NF = 100000
CH = 512
NCH = 196
NFP = CH * NCH
RB = 8

_U32 = jnp.uint32
_TINY = np.float32(np.finfo(np.float32).tiny)


def _threefry(x0, x1):
    k0 = np.uint32(0)
    k1 = np.uint32(42)
    k2 = np.uint32(k0 ^ k1 ^ np.uint32(0x1BD11BDA))
    ks = [k0, k1, k2]
    rots = [[13, 15, 26, 6], [17, 29, 16, 24]]

    def rotl(x, d):
        return (x << _U32(d)) | (x >> _U32(32 - d))

    x1 = x1 + k1
    for r in range(5):
        for d in rots[r % 2]:
            x0 = x0 + x1
            x1 = rotl(x1, d)
            x1 = x0 ^ x1
        x0 = x0 + ks[(r + 1) % 3]
        x1 = x1 + np.uint32(ks[(r + 2) % 3] + np.uint32(r + 1))
    return x0, x1


def _sample_body(l_ref, out_ref, *, nsamp):
    pid = pl.program_id(0)
    r0 = (pid * RB).astype(jnp.int32)
    srow = r0 + jax.lax.broadcasted_iota(jnp.int32, (RB, 1), 0)
    s_u = srow.astype(_U32)
    base_lo = s_u * _U32(np.uint32(NF))
    full = srow.astype(jnp.float32) * np.float32(NF)
    base_hi = (full >= np.float32(2.0 ** 32)).astype(_U32)
    iota_u = jax.lax.broadcasted_iota(_U32, (1, CH), 1)
    iota_i = jax.lax.broadcasted_iota(jnp.int32, (1, CH), 1)

    def body(c, carry):
        maxval, maxidx = carry
        f0 = c * CH
        x1c = base_lo + f0.astype(_U32)
        x1 = x1c + iota_u
        hi = base_hi + (x1 < base_lo).astype(_U32)
        o0, o1 = _threefry(hi, x1)
        bits = o0 ^ o1
        fb = (bits >> _U32(9)) | _U32(0x3F800000)
        u = jax.lax.bitcast_convert_type(fb, jnp.float32) - np.float32(1.0)
        u = jnp.maximum(u, _TINY)
        t = -jnp.log(-jnp.log(u)) + l_ref[0, c, :].reshape(1, CH)
        fv = f0 + iota_i
        upd = t > maxval
        maxval = jnp.where(upd, t, maxval)
        maxidx = jnp.where(upd, jnp.broadcast_to(fv, (RB, CH)), maxidx)
        return maxval, maxidx

    init = (jnp.full((RB, CH), -jnp.inf, jnp.float32),
            jnp.zeros((RB, CH), jnp.int32))
    maxval, maxidx = jax.lax.fori_loop(0, NCH, body, init)
    rm = jnp.max(maxval, axis=1, keepdims=True)
    cand = jnp.where(maxval == rm, maxidx, jnp.int32(1 << 30))
    out_ref[0, 0, :] = jnp.min(cand, axis=1)


def _face_sample(logits, nsamp):
    b = logits.shape[0]
    lp = jnp.full((b, NCH * CH), -jnp.inf, jnp.float32)
    lp = jax.lax.dynamic_update_slice(lp, logits, (0, 0))
    lp = lp.reshape(b, NCH, CH)
    nsteps = (b * nsamp) // RB
    steps_per_b = nsamp // RB
    out = pl.pallas_call(
        functools.partial(_sample_body, nsamp=nsamp),
        grid=(nsteps,),
        in_specs=[pl.BlockSpec((1, NCH, CH), lambda i: (i // steps_per_b, 0, 0))],
        out_specs=pl.BlockSpec((1, 1, RB), lambda i: (i, 0, 0)),
        out_shape=jax.ShapeDtypeStruct((nsteps, 1, RB), jnp.int32),
        compiler_params=pltpu.CompilerParams(
            dimension_semantics=("arbitrary",)),
    )(lp)
    return out.reshape(b, nsamp)


def kernel(V, F):
    b = V.shape[0]
    V0 = V[:, F[:, 0]]
    V01 = V[:, F[:, 1]] - V0
    V02 = V[:, F[:, 2]] - V0
    face_area = 0.5 * jnp.linalg.norm(jnp.cross(V01, V02, axis=-1), axis=-1)
    tot_area = jnp.sum(face_area, axis=-1, keepdims=True)
    face_prob = face_area / tot_area
    logits = jnp.log(face_prob + 1e-12)
    face_index = _face_sample(logits, NSAMP)
    batch_index = jnp.arange(b)[:, None]
    stacked = jnp.stack((V01, V02), axis=-1)
    samp_vecs = stacked[batch_index, face_index]
    samp_orig = V0[batch_index, face_index]
    key_r = jax.random.key(43)
    rand_scale = jax.random.uniform(key_r, (b, NSAMP, 2), dtype=jnp.float32)
    flip = jnp.sum(rand_scale, axis=-1) > 1.0
    rand_scale = jnp.where(flip[..., None], rand_scale - 1.0, rand_scale)
    rand_scale = jnp.abs(rand_scale)[:, :, None, :]
    samp_pts = samp_orig + jnp.sum(samp_vecs * rand_scale, axis=-1)
    return samp_pts


# speedup vs baseline: 1.5477x; 1.5477x over previous
import functools

import jax
import jax.numpy as jnp
import numpy as np
from jax.experimental import pallas as pl
from jax.experimental.pallas import tpu as pltpu

NSAMP = 8192
NF = 100000
CH = 1024
NCH = 98
NFP = CH * NCH
RB = 8

_U32 = jnp.uint32
_TINY = np.float32(np.finfo(np.float32).tiny)


def _threefry(x0, x1):
    k0 = np.uint32(0)
    k1 = np.uint32(42)
    k2 = np.uint32(k0 ^ k1 ^ np.uint32(0x1BD11BDA))
    ks = [k0, k1, k2]
    rots = [[13, 15, 26, 6], [17, 29, 16, 24]]

    def rotl(x, d):
        return (x << _U32(d)) | (x >> _U32(32 - d))

    x1 = x1 + k1
    for r in range(5):
        for d in rots[r % 2]:
            x0 = x0 + x1
            x1 = rotl(x1, d)
            x1 = x0 ^ x1
        x0 = x0 + ks[(r + 1) % 3]
        x1 = x1 + np.uint32(ks[(r + 2) % 3] + np.uint32(r + 1))
    return x0, x1


def _sample_body(l_ref, out_ref, *, nsamp):
    pid = pl.program_id(0)
    r0 = (pid * RB).astype(jnp.int32)
    srow = r0 + jax.lax.broadcasted_iota(jnp.int32, (RB, 1), 0)
    s_u = srow.astype(_U32)
    base_lo = s_u * _U32(np.uint32(NF))
    full = srow.astype(jnp.float32) * np.float32(NF)
    base_hi = (full >= np.float32(2.0 ** 32)).astype(_U32)
    iota_u = jax.lax.broadcasted_iota(_U32, (1, CH), 1)
    iota_i = jax.lax.broadcasted_iota(jnp.int32, (1, CH), 1)

    def body(c, carry):
        maxval, maxidx = carry
        f0 = c * CH
        x1c = base_lo + f0.astype(_U32)
        x1 = x1c + iota_u
        hi = base_hi + (x1 < base_lo).astype(_U32)
        o0, o1 = _threefry(hi, x1)
        bits = o0 ^ o1
        fb = (bits >> _U32(9)) | _U32(0x3F800000)
        u = jax.lax.bitcast_convert_type(fb, jnp.float32) - np.float32(1.0)
        u = jnp.maximum(u, _TINY)
        t = -jnp.log(-jnp.log(u)) + l_ref[0, c, :].reshape(1, CH)
        fv = f0 + iota_i
        upd = t > maxval
        maxval = jnp.where(upd, t, maxval)
        maxidx = jnp.where(upd, jnp.broadcast_to(fv, (RB, CH)), maxidx)
        return maxval, maxidx

    init = (jnp.full((RB, CH), -jnp.inf, jnp.float32),
            jnp.zeros((RB, CH), jnp.int32))
    maxval, maxidx = jax.lax.fori_loop(0, NCH, body, init)
    rm = jnp.max(maxval, axis=1, keepdims=True)
    cand = jnp.where(maxval == rm, maxidx, jnp.int32(1 << 30))
    out_ref[0, 0, :] = jnp.min(cand, axis=1)


def _face_sample(logits, nsamp):
    b = logits.shape[0]
    lp = jnp.full((b, NCH * CH), -jnp.inf, jnp.float32)
    lp = jax.lax.dynamic_update_slice(lp, logits, (0, 0))
    lp = lp.reshape(b, NCH, CH)
    nsteps = (b * nsamp) // RB
    steps_per_b = nsamp // RB
    out = pl.pallas_call(
        functools.partial(_sample_body, nsamp=nsamp),
        grid=(nsteps,),
        in_specs=[pl.BlockSpec((1, NCH, CH), lambda i: (i // steps_per_b, 0, 0))],
        out_specs=pl.BlockSpec((1, 1, RB), lambda i: (i, 0, 0)),
        out_shape=jax.ShapeDtypeStruct((nsteps, 1, RB), jnp.int32),
        compiler_params=pltpu.CompilerParams(
            dimension_semantics=("arbitrary",)),
    )(lp)
    return out.reshape(b, nsamp)


def kernel(V, F):
    b = V.shape[0]
    V0 = V[:, F[:, 0]]
    V01 = V[:, F[:, 1]] - V0
    V02 = V[:, F[:, 2]] - V0
    face_area = 0.5 * jnp.linalg.norm(jnp.cross(V01, V02, axis=-1), axis=-1)
    tot_area = jnp.sum(face_area, axis=-1, keepdims=True)
    face_prob = face_area / tot_area
    logits = jnp.log(face_prob + 1e-12)
    face_index = _face_sample(logits, NSAMP)
    batch_index = jnp.arange(b)[:, None]
    stacked = jnp.stack((V01, V02), axis=-1)
    samp_vecs = stacked[batch_index, face_index]
    samp_orig = V0[batch_index, face_index]
    key_r = jax.random.key(43)
    rand_scale = jax.random.uniform(key_r, (b, NSAMP, 2), dtype=jnp.float32)
    flip = jnp.sum(rand_scale, axis=-1) > 1.0
    rand_scale = jnp.where(flip[..., None], rand_scale - 1.0, rand_scale)
    rand_scale = jnp.abs(rand_scale)[:, :, None, :]
    samp_pts = samp_orig + jnp.sum(samp_vecs * rand_scale, axis=-1)
    return samp_pts


# speedup vs baseline: 1.5528x; 1.0032x over previous
import functools

import jax
import jax.numpy as jnp
import numpy as np
from jax.experimental import pallas as pl
from jax.experimental.pallas import tpu as pltpu

NSAMP = 8192
NF = 100000
CH = 1024
NCH = 98
NFP = CH * NCH
RB = 8

_U32 = jnp.uint32
_TINY = np.float32(np.finfo(np.float32).tiny)


def _threefry(x0, x1):
    k0 = np.uint32(0)
    k1 = np.uint32(42)
    k2 = np.uint32(k0 ^ k1 ^ np.uint32(0x1BD11BDA))
    ks = [k0, k1, k2]
    rots = [[13, 15, 26, 6], [17, 29, 16, 24]]

    def rotl(x, d):
        return (x << _U32(d)) | (x >> _U32(32 - d))

    x1 = x1 + k1
    for r in range(5):
        for d in rots[r % 2]:
            x0 = x0 + x1
            x1 = rotl(x1, d)
            x1 = x0 ^ x1
        x0 = x0 + ks[(r + 1) % 3]
        x1 = x1 + np.uint32(ks[(r + 2) % 3] + np.uint32(r + 1))
    return x0, x1


def _sample_body(l_ref, out_ref, *, nsamp):
    pid = pl.program_id(0)
    r0 = (pid * RB).astype(jnp.int32)
    srow = r0 + jax.lax.broadcasted_iota(jnp.int32, (RB, 1), 0)
    s_u = srow.astype(_U32)
    base_lo = s_u * _U32(np.uint32(NF))
    full = srow.astype(jnp.float32) * np.float32(NF)
    base_hi = (full >= np.float32(2.0 ** 32)).astype(_U32)
    iota_u = jax.lax.broadcasted_iota(_U32, (1, CH), 1)
    iota_i = jax.lax.broadcasted_iota(jnp.int32, (1, CH), 1)

    def body(c, carry):
        maxval, maxidx = carry
        f0 = c * CH
        x1c = base_lo + f0.astype(_U32)
        x1 = x1c + iota_u
        hi = base_hi + (x1 < base_lo).astype(_U32)
        o0, o1 = _threefry(hi, x1)
        bits = o0 ^ o1
        fb = (bits >> _U32(9)) | _U32(0x3F800000)
        u = jax.lax.bitcast_convert_type(fb, jnp.float32) - np.float32(1.0)
        t = -jnp.log(-jnp.log(u)) + l_ref[0, c, :].reshape(1, CH)
        upd = t > maxval
        maxval = jnp.where(upd, t, maxval)
        maxidx = jnp.where(upd, x1, maxidx)
        return maxval, maxidx

    init = (jnp.full((RB, CH), -jnp.inf, jnp.float32),
            jnp.zeros((RB, CH), _U32))
    maxval, maxidx = jax.lax.fori_loop(0, NCH, body, init)
    fidx = (maxidx - base_lo).astype(jnp.int32)
    rm = jnp.max(maxval, axis=1, keepdims=True)
    cand = jnp.where(maxval == rm, fidx, jnp.int32(1 << 30))
    out_ref[0, 0, :] = jnp.min(cand, axis=1)


def _face_sample(logits, nsamp):
    b = logits.shape[0]
    lp = jnp.full((b, NCH * CH), -jnp.inf, jnp.float32)
    lp = jax.lax.dynamic_update_slice(lp, logits, (0, 0))
    lp = lp.reshape(b, NCH, CH)
    nsteps = (b * nsamp) // RB
    steps_per_b = nsamp // RB
    out = pl.pallas_call(
        functools.partial(_sample_body, nsamp=nsamp),
        grid=(nsteps,),
        in_specs=[pl.BlockSpec((1, NCH, CH), lambda i: (i // steps_per_b, 0, 0))],
        out_specs=pl.BlockSpec((1, 1, RB), lambda i: (i, 0, 0)),
        out_shape=jax.ShapeDtypeStruct((nsteps, 1, RB), jnp.int32),
        compiler_params=pltpu.CompilerParams(
            dimension_semantics=("parallel",)),
    )(lp)
    return out.reshape(b, nsamp)


def kernel(V, F):
    b = V.shape[0]
    V0 = V[:, F[:, 0]]
    V01 = V[:, F[:, 1]] - V0
    V02 = V[:, F[:, 2]] - V0
    face_area = 0.5 * jnp.linalg.norm(jnp.cross(V01, V02, axis=-1), axis=-1)
    tot_area = jnp.sum(face_area, axis=-1, keepdims=True)
    face_prob = face_area / tot_area
    logits = jnp.log(face_prob + 1e-12)
    face_index = _face_sample(logits, NSAMP)
    batch_index = jnp.arange(b)[:, None]
    stacked = jnp.stack((V01, V02), axis=-1)
    samp_vecs = stacked[batch_index, face_index]
    samp_orig = V0[batch_index, face_index]
    key_r = jax.random.key(43)
    rand_scale = jax.random.uniform(key_r, (b, NSAMP, 2), dtype=jnp.float32)
    flip = jnp.sum(rand_scale, axis=-1) > 1.0
    rand_scale = jnp.where(flip[..., None], rand_scale - 1.0, rand_scale)
    rand_scale = jnp.abs(rand_scale)[:, :, None, :]
    samp_pts = samp_orig + jnp.sum(samp_vecs * rand_scale, axis=-1)
    return samp_pts


# speedup vs baseline: 1.5616x; 1.0057x over previous
import functools

import jax
import jax.numpy as jnp
import numpy as np
from jax.experimental import pallas as pl
from jax.experimental.pallas import tpu as pltpu

NSAMP = 8192
NF = 100000
CH = 1024
NCH = 98
NFP = CH * NCH
RB = 8

_U32 = jnp.uint32
_TINY = np.float32(np.finfo(np.float32).tiny)


def _threefry(x0, x1):
    k0 = np.uint32(0)
    k1 = np.uint32(42)
    k2 = np.uint32(k0 ^ k1 ^ np.uint32(0x1BD11BDA))
    ks = [k0, k1, k2]
    rots = [[13, 15, 26, 6], [17, 29, 16, 24]]

    def rotl(x, d):
        return (x << _U32(d)) | (x >> _U32(32 - d))

    x1 = x1 + k1
    for r in range(5):
        for d in rots[r % 2]:
            x0 = x0 + x1
            x1 = rotl(x1, d)
            x1 = x0 ^ x1
        x0 = x0 + ks[(r + 1) % 3]
        x1 = x1 + np.uint32(ks[(r + 2) % 3] + np.uint32(r + 1))
    return x0, x1


GROUPS = 4


def _sample_body(l_ref, out_ref, *, nsamp):
    pid = pl.program_id(0)
    for g in range(GROUPS):
        _sample_group(l_ref, out_ref, pid * (RB * GROUPS) + g * RB, g)


def _sample_group(l_ref, out_ref, r0, g):
    srow = r0 + jax.lax.broadcasted_iota(jnp.int32, (RB, 1), 0)
    s_u = srow.astype(_U32)
    base_lo = s_u * _U32(np.uint32(NF))
    full = srow.astype(jnp.float32) * np.float32(NF)
    base_hi = (full >= np.float32(2.0 ** 32)).astype(_U32)
    iota_u = jax.lax.broadcasted_iota(_U32, (1, CH), 1)
    iota_i = jax.lax.broadcasted_iota(jnp.int32, (1, CH), 1)

    def body(c, carry):
        maxval, maxidx = carry
        f0 = c * CH
        x1c = base_lo + f0.astype(_U32)
        x1 = x1c + iota_u
        hi = base_hi + (x1 < base_lo).astype(_U32)
        o0, o1 = _threefry(hi, x1)
        bits = o0 ^ o1
        fb = (bits >> _U32(9)) | _U32(0x3F800000)
        u = jax.lax.bitcast_convert_type(fb, jnp.float32) - np.float32(1.0)
        t = -jnp.log(-jnp.log(u)) + l_ref[0, c, :].reshape(1, CH)
        upd = t > maxval
        maxval = jnp.where(upd, t, maxval)
        maxidx = jnp.where(upd, x1, maxidx)
        return maxval, maxidx

    init = (jnp.full((RB, CH), -jnp.inf, jnp.float32),
            jnp.zeros((RB, CH), _U32))
    maxval, maxidx = jax.lax.fori_loop(0, NCH, body, init)
    fidx = (maxidx - base_lo).astype(jnp.int32)
    rm = jnp.max(maxval, axis=1, keepdims=True)
    cand = jnp.where(maxval == rm, fidx, jnp.int32(1 << 30))
    out_ref[0, g, :] = jnp.min(cand, axis=1)


def _face_sample(logits, nsamp):
    b = logits.shape[0]
    lp = jnp.full((b, NCH * CH), -jnp.inf, jnp.float32)
    lp = jax.lax.dynamic_update_slice(lp, logits, (0, 0))
    lp = lp.reshape(b, NCH, CH)
    nsteps = (b * nsamp) // (RB * GROUPS)
    steps_per_b = nsamp // (RB * GROUPS)
    out = pl.pallas_call(
        functools.partial(_sample_body, nsamp=nsamp),
        grid=(nsteps,),
        in_specs=[pl.BlockSpec((1, NCH, CH), lambda i: (i // steps_per_b, 0, 0))],
        out_specs=pl.BlockSpec((1, GROUPS, RB), lambda i: (i, 0, 0)),
        out_shape=jax.ShapeDtypeStruct((nsteps, GROUPS, RB), jnp.int32),
        compiler_params=pltpu.CompilerParams(
            dimension_semantics=("parallel",)),
    )(lp)
    return out.reshape(b, nsamp)


def kernel(V, F):
    b = V.shape[0]
    V0 = V[:, F[:, 0]]
    V01 = V[:, F[:, 1]] - V0
    V02 = V[:, F[:, 2]] - V0
    face_area = 0.5 * jnp.linalg.norm(jnp.cross(V01, V02, axis=-1), axis=-1)
    tot_area = jnp.sum(face_area, axis=-1, keepdims=True)
    face_prob = face_area / tot_area
    logits = jnp.log(face_prob + 1e-12)
    face_index = _face_sample(logits, NSAMP)
    batch_index = jnp.arange(b)[:, None]
    stacked = jnp.stack((V01, V02), axis=-1)
    samp_vecs = stacked[batch_index, face_index]
    samp_orig = V0[batch_index, face_index]
    key_r = jax.random.key(43)
    rand_scale = jax.random.uniform(key_r, (b, NSAMP, 2), dtype=jnp.float32)
    flip = jnp.sum(rand_scale, axis=-1) > 1.0
    rand_scale = jnp.where(flip[..., None], rand_scale - 1.0, rand_scale)
    rand_scale = jnp.abs(rand_scale)[:, :, None, :]
    samp_pts = samp_orig + jnp.sum(samp_vecs * rand_scale, axis=-1)
    return samp_pts


# speedup vs baseline: 1.7068x; 1.0929x over previous
import functools

import jax
import jax.numpy as jnp
import numpy as np
from jax.experimental import pallas as pl
from jax.experimental.pallas import tpu as pltpu

NSAMP = 8192
NF = 100000
CH = 1024
NCH = 98
NFP = CH * NCH
RB = 8

_U32 = jnp.uint32
_TINY = np.float32(np.finfo(np.float32).tiny)


def _threefry(x0, x1):
    k0 = np.uint32(0)
    k1 = np.uint32(42)
    k2 = np.uint32(k0 ^ k1 ^ np.uint32(0x1BD11BDA))
    ks = [k0, k1, k2]
    rots = [[13, 15, 26, 6], [17, 29, 16, 24]]

    def rotl(x, d):
        return (x << _U32(d)) | (x >> _U32(32 - d))

    x1 = x1 + k1
    for r in range(5):
        for d in rots[r % 2]:
            x0 = x0 + x1
            x1 = rotl(x1, d)
            x1 = x0 ^ x1
        x0 = x0 + ks[(r + 1) % 3]
        x1 = x1 + np.uint32(ks[(r + 2) % 3] + np.uint32(r + 1))
    return x0, x1


GROUPS = 4


def _sample_body(l_ref, out_ref, *, nsamp):
    pid = pl.program_id(0)
    for g in range(GROUPS):
        _sample_group(l_ref, out_ref, pid * (RB * GROUPS) + g * RB, g)


def _sample_group(l_ref, out_ref, r0, g):
    srow = r0 + jax.lax.broadcasted_iota(jnp.int32, (RB, 1), 0)
    s_u = srow.astype(_U32)
    base_lo = s_u * _U32(np.uint32(NF))
    full = srow.astype(jnp.float32) * np.float32(NF)
    base_hi = (full >= np.float32(2.0 ** 32)).astype(_U32)
    iota_u = jax.lax.broadcasted_iota(_U32, (1, CH), 1)
    iota_i = jax.lax.broadcasted_iota(jnp.int32, (1, CH), 1)

    def body(c, carry):
        maxval, maxidx = carry
        f0 = c * CH
        x1c = base_lo + f0.astype(_U32)
        x1 = x1c + iota_u
        hi = base_hi + (x1 < base_lo).astype(_U32)
        o0, o1 = _threefry(hi, x1)
        bits = o0 ^ o1
        fb = (bits >> _U32(9)) | _U32(0x3F800000)
        u = jax.lax.bitcast_convert_type(fb, jnp.float32) - np.float32(1.0)
        t = -jnp.log(-jnp.log(u)) + l_ref[0, c, :].reshape(1, CH)
        upd = t > maxval
        maxval = jnp.where(upd, t, maxval)
        maxidx = jnp.where(upd, x1, maxidx)
        return maxval, maxidx

    init = (jnp.full((RB, CH), -jnp.inf, jnp.float32),
            jnp.zeros((RB, CH), _U32))
    maxval, maxidx = jax.lax.fori_loop(0, NCH, body, init, unroll=7)
    fidx = (maxidx - base_lo).astype(jnp.int32)
    rm = jnp.max(maxval, axis=1, keepdims=True)
    cand = jnp.where(maxval == rm, fidx, jnp.int32(1 << 30))
    out_ref[0, g, :] = jnp.min(cand, axis=1)


def _face_sample(logits, nsamp):
    b = logits.shape[0]
    lp = jnp.full((b, NCH * CH), -jnp.inf, jnp.float32)
    lp = jax.lax.dynamic_update_slice(lp, logits, (0, 0))
    lp = lp.reshape(b, NCH, CH)
    nsteps = (b * nsamp) // (RB * GROUPS)
    steps_per_b = nsamp // (RB * GROUPS)
    out = pl.pallas_call(
        functools.partial(_sample_body, nsamp=nsamp),
        grid=(nsteps,),
        in_specs=[pl.BlockSpec((1, NCH, CH), lambda i: (i // steps_per_b, 0, 0))],
        out_specs=pl.BlockSpec((1, GROUPS, RB), lambda i: (i, 0, 0)),
        out_shape=jax.ShapeDtypeStruct((nsteps, GROUPS, RB), jnp.int32),
        compiler_params=pltpu.CompilerParams(
            dimension_semantics=("parallel",)),
    )(lp)
    return out.reshape(b, nsamp)


def kernel(V, F):
    b = V.shape[0]
    V0 = V[:, F[:, 0]]
    V01 = V[:, F[:, 1]] - V0
    V02 = V[:, F[:, 2]] - V0
    face_area = 0.5 * jnp.linalg.norm(jnp.cross(V01, V02, axis=-1), axis=-1)
    tot_area = jnp.sum(face_area, axis=-1, keepdims=True)
    face_prob = face_area / tot_area
    logits = jnp.log(face_prob + 1e-12)
    face_index = _face_sample(logits, NSAMP)
    batch_index = jnp.arange(b)[:, None]
    stacked = jnp.stack((V01, V02), axis=-1)
    samp_vecs = stacked[batch_index, face_index]
    samp_orig = V0[batch_index, face_index]
    key_r = jax.random.key(43)
    rand_scale = jax.random.uniform(key_r, (b, NSAMP, 2), dtype=jnp.float32)
    flip = jnp.sum(rand_scale, axis=-1) > 1.0
    rand_scale = jnp.where(flip[..., None], rand_scale - 1.0, rand_scale)
    rand_scale = jnp.abs(rand_scale)[:, :, None, :]
    samp_pts = samp_orig + jnp.sum(samp_vecs * rand_scale, axis=-1)
    return samp_pts


# speedup vs baseline: 1.7122x; 1.0032x over previous
import functools

import jax
import jax.numpy as jnp
import numpy as np
from jax.experimental import pallas as pl
from jax.experimental.pallas import tpu as pltpu

NSAMP = 8192
NF = 100000
CH = 1024
NCH = 98
NFP = CH * NCH
RB = 8

_U32 = jnp.uint32
_TINY = np.float32(np.finfo(np.float32).tiny)


def _threefry(x0, x1):
    k0 = np.uint32(0)
    k1 = np.uint32(42)
    k2 = np.uint32(k0 ^ k1 ^ np.uint32(0x1BD11BDA))
    ks = [k0, k1, k2]
    rots = [[13, 15, 26, 6], [17, 29, 16, 24]]

    def rotl(x, d):
        return (x << _U32(d)) | (x >> _U32(32 - d))

    x1 = x1 + k1
    for r in range(5):
        for d in rots[r % 2]:
            x0 = x0 + x1
            x1 = rotl(x1, d)
            x1 = x0 ^ x1
        x0 = x0 + ks[(r + 1) % 3]
        x1 = x1 + np.uint32(ks[(r + 2) % 3] + np.uint32(r + 1))
    return x0, x1


GROUPS = 4


def _sample_body(l_ref, out_ref, *, nsamp):
    pid = pl.program_id(0)
    for g in range(GROUPS):
        _sample_group(l_ref, out_ref, pid * (RB * GROUPS) + g * RB, g)


def _sample_group(l_ref, out_ref, r0, g):
    srow = r0 + jax.lax.broadcasted_iota(jnp.int32, (RB, 1), 0)
    s_u = srow.astype(_U32)
    base_lo = s_u * _U32(np.uint32(NF))
    full = srow.astype(jnp.float32) * np.float32(NF)
    base_hi = (full >= np.float32(2.0 ** 32)).astype(_U32)
    iota_u = jax.lax.broadcasted_iota(_U32, (1, CH), 1)
    iota_i = jax.lax.broadcasted_iota(jnp.int32, (1, CH), 1)

    def body(c, carry):
        maxval, maxidx = carry
        f0 = c * CH
        x1c = base_lo + f0.astype(_U32)
        x1 = x1c + iota_u
        hi = base_hi + (x1 < base_lo).astype(_U32)
        o0, o1 = _threefry(hi, x1)
        bits = o0 ^ o1
        fb = (bits >> _U32(9)) | _U32(0x3F800000)
        u = jax.lax.bitcast_convert_type(fb, jnp.float32) - np.float32(1.0)
        t = -jnp.log(-jnp.log(u)) + l_ref[0, c, :].reshape(1, CH)
        upd = t > maxval
        maxval = jnp.where(upd, t, maxval)
        maxidx = jnp.where(upd, x1, maxidx)
        return maxval, maxidx

    init = (jnp.full((RB, CH), -jnp.inf, jnp.float32),
            jnp.zeros((RB, CH), _U32))
    maxval, maxidx = jax.lax.fori_loop(0, NCH, body, init, unroll=14)
    fidx = (maxidx - base_lo).astype(jnp.int32)
    rm = jnp.max(maxval, axis=1, keepdims=True)
    cand = jnp.where(maxval == rm, fidx, jnp.int32(1 << 30))
    out_ref[0, g, :] = jnp.min(cand, axis=1)


def _face_sample(logits, nsamp):
    b = logits.shape[0]
    lp = jnp.full((b, NCH * CH), -jnp.inf, jnp.float32)
    lp = jax.lax.dynamic_update_slice(lp, logits, (0, 0))
    lp = lp.reshape(b, NCH, CH)
    nsteps = (b * nsamp) // (RB * GROUPS)
    steps_per_b = nsamp // (RB * GROUPS)
    out = pl.pallas_call(
        functools.partial(_sample_body, nsamp=nsamp),
        grid=(nsteps,),
        in_specs=[pl.BlockSpec((1, NCH, CH), lambda i: (i // steps_per_b, 0, 0))],
        out_specs=pl.BlockSpec((1, GROUPS, RB), lambda i: (i, 0, 0)),
        out_shape=jax.ShapeDtypeStruct((nsteps, GROUPS, RB), jnp.int32),
        compiler_params=pltpu.CompilerParams(
            dimension_semantics=("parallel",)),
    )(lp)
    return out.reshape(b, nsamp)


def kernel(V, F):
    b = V.shape[0]
    V0 = V[:, F[:, 0]]
    V01 = V[:, F[:, 1]] - V0
    V02 = V[:, F[:, 2]] - V0
    face_area = 0.5 * jnp.linalg.norm(jnp.cross(V01, V02, axis=-1), axis=-1)
    tot_area = jnp.sum(face_area, axis=-1, keepdims=True)
    face_prob = face_area / tot_area
    logits = jnp.log(face_prob + 1e-12)
    face_index = _face_sample(logits, NSAMP)
    batch_index = jnp.arange(b)[:, None]
    stacked = jnp.stack((V01, V02), axis=-1)
    samp_vecs = stacked[batch_index, face_index]
    samp_orig = V0[batch_index, face_index]
    key_r = jax.random.key(43)
    rand_scale = jax.random.uniform(key_r, (b, NSAMP, 2), dtype=jnp.float32)
    flip = jnp.sum(rand_scale, axis=-1) > 1.0
    rand_scale = jnp.where(flip[..., None], rand_scale - 1.0, rand_scale)
    rand_scale = jnp.abs(rand_scale)[:, :, None, :]
    samp_pts = samp_orig + jnp.sum(samp_vecs * rand_scale, axis=-1)
    return samp_pts


# speedup vs baseline: 1.7404x; 1.0164x over previous
import functools

import jax
import jax.numpy as jnp
import numpy as np
from jax.experimental import pallas as pl
from jax.experimental.pallas import tpu as pltpu

NSAMP = 8192
NF = 100000
CH = 1024
NCH = 98
NFP = CH * NCH
RB = 8

_U32 = jnp.uint32
_TINY = np.float32(np.finfo(np.float32).tiny)


def _threefry(x0, x1, key1=42):
    k0 = np.uint32(0)
    k1 = np.uint32(key1)
    k2 = np.uint32(k0 ^ k1 ^ np.uint32(0x1BD11BDA))
    ks = [k0, k1, k2]
    rots = [[13, 15, 26, 6], [17, 29, 16, 24]]

    def rotl(x, d):
        return (x << _U32(d)) | (x >> _U32(32 - d))

    x1 = x1 + k1
    for r in range(5):
        for d in rots[r % 2]:
            x0 = x0 + x1
            x1 = rotl(x1, d)
            x1 = x0 ^ x1
        x0 = x0 + ks[(r + 1) % 3]
        x1 = x1 + np.uint32(ks[(r + 2) % 3] + np.uint32(r + 1))
    return x0, x1


GROUPS = 4


def _sample_body(l_ref, out_ref, *, nsamp):
    pid = pl.program_id(0)
    for g in range(GROUPS):
        _sample_group(l_ref, out_ref, pid * (RB * GROUPS) + g * RB, g)


def _sample_group(l_ref, out_ref, r0, g):
    srow = r0 + jax.lax.broadcasted_iota(jnp.int32, (RB, 1), 0)
    s_u = srow.astype(_U32)
    base_lo = s_u * _U32(np.uint32(NF))
    full = srow.astype(jnp.float32) * np.float32(NF)
    base_hi = (full >= np.float32(2.0 ** 32)).astype(_U32)
    iota_u = jax.lax.broadcasted_iota(_U32, (1, CH), 1)
    iota_i = jax.lax.broadcasted_iota(jnp.int32, (1, CH), 1)

    def body(c, carry):
        maxval, maxidx = carry
        f0 = c * CH
        x1c = base_lo + f0.astype(_U32)
        x1 = x1c + iota_u
        hi = base_hi + (x1 < base_lo).astype(_U32)
        o0, o1 = _threefry(hi, x1)
        bits = o0 ^ o1
        fb = (bits >> _U32(9)) | _U32(0x3F800000)
        u = jax.lax.bitcast_convert_type(fb, jnp.float32) - np.float32(1.0)
        t = -jnp.log(-jnp.log(u)) + l_ref[0, c, :].reshape(1, CH)
        upd = t > maxval
        maxval = jnp.where(upd, t, maxval)
        maxidx = jnp.where(upd, x1, maxidx)
        return maxval, maxidx

    init = (jnp.full((RB, CH), -jnp.inf, jnp.float32),
            jnp.zeros((RB, CH), _U32))
    maxval, maxidx = jax.lax.fori_loop(0, NCH, body, init, unroll=14)
    fidx = (maxidx - base_lo).astype(jnp.int32)
    rm = jnp.max(maxval, axis=1, keepdims=True)
    cand = jnp.where(maxval == rm, fidx, jnp.int32(1 << 30))
    out_ref[0, g, :] = jnp.min(cand, axis=1)


def _face_sample(logits, nsamp):
    b = logits.shape[0]
    lp = jnp.full((b, NCH * CH), -jnp.inf, jnp.float32)
    lp = jax.lax.dynamic_update_slice(lp, logits, (0, 0))
    lp = lp.reshape(b, NCH, CH)
    nsteps = (b * nsamp) // (RB * GROUPS)
    steps_per_b = nsamp // (RB * GROUPS)
    out = pl.pallas_call(
        functools.partial(_sample_body, nsamp=nsamp),
        grid=(nsteps,),
        in_specs=[pl.BlockSpec((1, NCH, CH), lambda i: (i // steps_per_b, 0, 0))],
        out_specs=pl.BlockSpec((1, GROUPS, RB), lambda i: (i, 0, 0)),
        out_shape=jax.ShapeDtypeStruct((nsteps, GROUPS, RB), jnp.int32),
        compiler_params=pltpu.CompilerParams(
            dimension_semantics=("parallel",)),
    )(lp)
    return out.reshape(b, nsamp)


def _w_body(o0_ref, o1_ref):
    b = pl.program_id(0)
    sv = jax.lax.broadcasted_iota(_U32, (1, NSAMP), 1)
    base = (_U32(b * NSAMP * 2) + sv * _U32(2))
    z = jnp.zeros_like(base)
    for j, ref in ((0, o0_ref), (1, o1_ref)):
        a0, a1 = _threefry(z, base + _U32(j), key1=43)
        bits = a0 ^ a1
        fb = (bits >> _U32(9)) | _U32(0x3F800000)
        ref[0] = jax.lax.bitcast_convert_type(fb, jnp.float32) - np.float32(1.0)


def _rand_scale_pair():
    return pl.pallas_call(
        _w_body,
        grid=(8,),
        in_specs=[],
        out_specs=[pl.BlockSpec((1, 1, NSAMP), lambda i: (i, 0, 0))] * 2,
        out_shape=[jax.ShapeDtypeStruct((8, 1, NSAMP), jnp.float32)] * 2,
    )()


def _geom_body(a_ref, b_ref, out_ref):
    a = a_ref[0]
    bb = b_ref[0]
    cx = a[1:2] * bb[2:3] - a[2:3] * bb[1:2]
    cy = a[2:3] * bb[0:1] - a[0:1] * bb[2:3]
    cz = a[0:1] * bb[1:2] - a[1:2] * bb[0:1]
    q = cx * cx + cy * cy + cz * cz
    area = np.float32(0.5) * jnp.sqrt(q)
    tot = jnp.sum(area, axis=1, keepdims=True)
    lg = jnp.log(area / tot + np.float32(1e-12))
    lane = jax.lax.broadcasted_iota(jnp.int32, (1, NFP), 1)
    out_ref[0] = jnp.where(lane < NF, lg, -jnp.inf)


def _face_logits(V01T, V02T):
    return pl.pallas_call(
        _geom_body,
        grid=(8,),
        in_specs=[pl.BlockSpec((1, 3, NFP), lambda i: (i, 0, 0))] * 2,
        out_specs=pl.BlockSpec((1, 1, NFP), lambda i: (i, 0, 0)),
        out_shape=jax.ShapeDtypeStruct((8, 1, NFP), jnp.float32),
    )(V01T, V02T).reshape(8, NFP)


def _make_epilogue():
    from jax import lax
    from jax.experimental.pallas import tpu_sc as plsc

    mesh = plsc.VectorSubcoreMesh(core_axis_name="c", subcore_axis_name="s")
    CS = (8 * NSAMP) // 32

    @functools.partial(
        pl.kernel, mesh=mesh,
        out_type=jax.ShapeDtypeStruct((3, 32, CS), jnp.float32),
        scratch_types=[
            pltpu.VMEM((CS,), jnp.int32),
            pltpu.VMEM((CS,), jnp.float32),
            pltpu.VMEM((CS,), jnp.float32),
            pltpu.VMEM((CS,), jnp.float32),
            pltpu.VMEM((CS,), jnp.float32),
            pltpu.VMEM((CS,), jnp.float32),
            pltpu.VMEM((CS,), jnp.float32),
            pltpu.SemaphoreType.DMA,
        ],
    )
    def epi(colflat_hbm, gidx9_hbm, w0_hbm, w1_hbm, out_hbm,
            idx_v, buf_v, w0_v, w1_v, ox_v, oy_v, oz_v, sem):
        wid = lax.axis_index("s") * 2 + lax.axis_index("c")
        pltpu.sync_copy(w0_hbm.at[wid], w0_v)
        pltpu.sync_copy(w1_hbm.at[wid], w1_v)
        outs = (ox_v, oy_v, oz_v)
        for t in range(9):
            k, c = divmod(t, 3)
            pltpu.sync_copy(gidx9_hbm.at[t, wid], idx_v)
            pltpu.async_copy(colflat_hbm.at[idx_v], buf_v, sem).wait()
            tgt = outs[c]
            wv = (None, w0_v, w1_v)[k]

            def chunk(i, _):
                sl = pl.ds(i * 16, 16)
                val = buf_v[sl]
                if wv is None:
                    tgt[sl] = val
                else:
                    tgt[sl] = tgt[sl] + wv[sl] * val
                return ()

            jax.lax.fori_loop(0, CS // 16, chunk, ())
        pltpu.sync_copy(ox_v, out_hbm.at[0, wid])
        pltpu.sync_copy(oy_v, out_hbm.at[1, wid])
        pltpu.sync_copy(oz_v, out_hbm.at[2, wid])

    return epi


def kernel(V, F):
    b = V.shape[0]
    V0 = V[:, F[:, 0]]
    V01 = V[:, F[:, 1]] - V0
    V02 = V[:, F[:, 2]] - V0
    pad = ((0, 0), (0, NFP - NF), (0, 0))
    v01p = jnp.pad(V01, pad).transpose(0, 2, 1)
    v02p = jnp.pad(V02, pad).transpose(0, 2, 1)
    logits = _face_logits(v01p, v02p)
    face_index = _face_sample_padded(logits)
    w0, w1 = _rand_scale_pair()
    w0 = w0.reshape(b, NSAMP)
    w1 = w1.reshape(b, NSAMP)
    flip = (w0 + w1) > 1.0
    w0 = jnp.abs(jnp.where(flip, w0 - 1.0, w0))
    w1 = jnp.abs(jnp.where(flip, w1 - 1.0, w1))
    colflat = jnp.stack([
        jnp.stack([Vk[:, :, c] for c in range(3)])
        for Vk in (V0, V01, V02)]).reshape(-1)
    gidx = face_index + (jnp.arange(b, dtype=jnp.int32) * NF)[:, None]
    gidx9 = (gidx[None] +
             (jnp.arange(9, dtype=jnp.int32) * (b * NF))[:, None, None])
    gidx9 = gidx9.reshape(9, 32, -1)
    out = _make_epilogue()(colflat, gidx9, w0.reshape(32, -1), w1.reshape(32, -1))
    return jnp.moveaxis(out.reshape(3, b, NSAMP), 0, -1)


def _face_sample_padded(lp):
    lp = lp.reshape(8, NCH, CH)
    nsteps = (8 * NSAMP) // (RB * GROUPS)
    steps_per_b = NSAMP // (RB * GROUPS)
    out = pl.pallas_call(
        functools.partial(_sample_body, nsamp=NSAMP),
        grid=(nsteps,),
        in_specs=[pl.BlockSpec((1, NCH, CH), lambda i: (i // steps_per_b, 0, 0))],
        out_specs=pl.BlockSpec((1, GROUPS, RB), lambda i: (i, 0, 0)),
        out_shape=jax.ShapeDtypeStruct((nsteps, GROUPS, RB), jnp.int32),
        compiler_params=pltpu.CompilerParams(
            dimension_semantics=("parallel",)),
    )(lp)
    return out.reshape(8, NSAMP)


# speedup vs baseline: 1.7444x; 1.0023x over previous
import functools

import jax
import jax.numpy as jnp
import numpy as np
from jax.experimental import pallas as pl
from jax.experimental.pallas import tpu as pltpu

NSAMP = 8192
NF = 100000
CH = 1024
NCH = 98
NFP = CH * NCH
RB = 8

_U32 = jnp.uint32
_TINY = np.float32(np.finfo(np.float32).tiny)


def _threefry(x0, x1, key1=42):
    k0 = np.uint32(0)
    k1 = np.uint32(key1)
    k2 = np.uint32(k0 ^ k1 ^ np.uint32(0x1BD11BDA))
    ks = [k0, k1, k2]
    rots = [[13, 15, 26, 6], [17, 29, 16, 24]]

    def rotl(x, d):
        return (x << _U32(d)) | (x >> _U32(32 - d))

    x1 = x1 + k1
    for r in range(5):
        for d in rots[r % 2]:
            x0 = x0 + x1
            x1 = rotl(x1, d)
            x1 = x0 ^ x1
        x0 = x0 + ks[(r + 1) % 3]
        x1 = x1 + np.uint32(ks[(r + 2) % 3] + np.uint32(r + 1))
    return x0, x1


GROUPS = 4


def _sample_body(l_ref, out_ref, *, nsamp):
    pid = pl.program_id(0)
    for g in range(GROUPS):
        _sample_group(l_ref, out_ref, pid * (RB * GROUPS) + g * RB, g)


def _sample_group(l_ref, out_ref, r0, g):
    srow = r0 + jax.lax.broadcasted_iota(jnp.int32, (RB, 1), 0)
    s_u = srow.astype(_U32)
    base_lo = s_u * _U32(np.uint32(NF))
    full = srow.astype(jnp.float32) * np.float32(NF)
    base_hi = (full >= np.float32(2.0 ** 32)).astype(_U32)
    iota_u = jax.lax.broadcasted_iota(_U32, (1, CH), 1)
    iota_i = jax.lax.broadcasted_iota(jnp.int32, (1, CH), 1)

    def body(c, carry):
        maxval, maxidx = carry
        f0 = c * CH
        x1c = base_lo + f0.astype(_U32)
        x1 = x1c + iota_u
        hi = base_hi + (x1 < base_lo).astype(_U32)
        o0, o1 = _threefry(hi, x1)
        bits = o0 ^ o1
        fb = (bits >> _U32(9)) | _U32(0x3F800000)
        u = jax.lax.bitcast_convert_type(fb, jnp.float32) - np.float32(1.0)
        t = -jnp.log(-jnp.log(u)) + l_ref[0, c, :].reshape(1, CH)
        upd = t > maxval
        maxval = jnp.where(upd, t, maxval)
        maxidx = jnp.where(upd, x1, maxidx)
        return maxval, maxidx

    init = (jnp.full((RB, CH), -jnp.inf, jnp.float32),
            jnp.zeros((RB, CH), _U32))
    maxval, maxidx = jax.lax.fori_loop(0, NCH, body, init, unroll=28)
    fidx = (maxidx - base_lo).astype(jnp.int32)
    rm = jnp.max(maxval, axis=1, keepdims=True)
    cand = jnp.where(maxval == rm, fidx, jnp.int32(1 << 30))
    out_ref[0, g, :] = jnp.min(cand, axis=1)


def _face_sample(logits, nsamp):
    b = logits.shape[0]
    lp = jnp.full((b, NCH * CH), -jnp.inf, jnp.float32)
    lp = jax.lax.dynamic_update_slice(lp, logits, (0, 0))
    lp = lp.reshape(b, NCH, CH)
    nsteps = (b * nsamp) // (RB * GROUPS)
    steps_per_b = nsamp // (RB * GROUPS)
    out = pl.pallas_call(
        functools.partial(_sample_body, nsamp=nsamp),
        grid=(nsteps,),
        in_specs=[pl.BlockSpec((1, NCH, CH), lambda i: (i // steps_per_b, 0, 0))],
        out_specs=pl.BlockSpec((1, GROUPS, RB), lambda i: (i, 0, 0)),
        out_shape=jax.ShapeDtypeStruct((nsteps, GROUPS, RB), jnp.int32),
        compiler_params=pltpu.CompilerParams(
            dimension_semantics=("parallel",)),
    )(lp)
    return out.reshape(b, nsamp)


def _w_body(o0_ref, o1_ref):
    b = pl.program_id(0)
    sv = jax.lax.broadcasted_iota(_U32, (1, NSAMP), 1)
    base = (_U32(b * NSAMP * 2) + sv * _U32(2))
    z = jnp.zeros_like(base)
    for j, ref in ((0, o0_ref), (1, o1_ref)):
        a0, a1 = _threefry(z, base + _U32(j), key1=43)
        bits = a0 ^ a1
        fb = (bits >> _U32(9)) | _U32(0x3F800000)
        ref[0] = jax.lax.bitcast_convert_type(fb, jnp.float32) - np.float32(1.0)


def _rand_scale_pair():
    return pl.pallas_call(
        _w_body,
        grid=(8,),
        in_specs=[],
        out_specs=[pl.BlockSpec((1, 1, NSAMP), lambda i: (i, 0, 0))] * 2,
        out_shape=[jax.ShapeDtypeStruct((8, 1, NSAMP), jnp.float32)] * 2,
    )()


def _geom_body(a_ref, b_ref, out_ref):
    a = a_ref[0]
    bb = b_ref[0]
    cx = a[1:2] * bb[2:3] - a[2:3] * bb[1:2]
    cy = a[2:3] * bb[0:1] - a[0:1] * bb[2:3]
    cz = a[0:1] * bb[1:2] - a[1:2] * bb[0:1]
    q = cx * cx + cy * cy + cz * cz
    area = np.float32(0.5) * jnp.sqrt(q)
    tot = jnp.sum(area, axis=1, keepdims=True)
    lg = jnp.log(area / tot + np.float32(1e-12))
    lane = jax.lax.broadcasted_iota(jnp.int32, (1, NFP), 1)
    out_ref[0] = jnp.where(lane < NF, lg, -jnp.inf)


def _face_logits(V01T, V02T):
    return pl.pallas_call(
        _geom_body,
        grid=(8,),
        in_specs=[pl.BlockSpec((1, 3, NFP), lambda i: (i, 0, 0))] * 2,
        out_specs=pl.BlockSpec((1, 1, NFP), lambda i: (i, 0, 0)),
        out_shape=jax.ShapeDtypeStruct((8, 1, NFP), jnp.float32),
    )(V01T, V02T).reshape(8, NFP)


def _make_epilogue():
    from jax import lax
    from jax.experimental.pallas import tpu_sc as plsc

    mesh = plsc.VectorSubcoreMesh(core_axis_name="c", subcore_axis_name="s")
    CS = (8 * NSAMP) // 32

    @functools.partial(
        pl.kernel, mesh=mesh,
        out_type=jax.ShapeDtypeStruct((3, 32, CS), jnp.float32),
        scratch_types=[
            pltpu.VMEM((CS,), jnp.int32),
            pltpu.VMEM((CS,), jnp.float32),
            pltpu.VMEM((CS,), jnp.float32),
            pltpu.VMEM((CS,), jnp.float32),
            pltpu.VMEM((CS,), jnp.float32),
            pltpu.VMEM((CS,), jnp.float32),
            pltpu.VMEM((CS,), jnp.float32),
            pltpu.SemaphoreType.DMA,
        ],
    )
    def epi(colflat_hbm, gidx9_hbm, w0_hbm, w1_hbm, out_hbm,
            idx_v, buf_v, w0_v, w1_v, ox_v, oy_v, oz_v, sem):
        wid = lax.axis_index("s") * 2 + lax.axis_index("c")
        pltpu.sync_copy(w0_hbm.at[wid], w0_v)
        pltpu.sync_copy(w1_hbm.at[wid], w1_v)
        outs = (ox_v, oy_v, oz_v)
        for t in range(9):
            k, c = divmod(t, 3)
            pltpu.sync_copy(gidx9_hbm.at[t, wid], idx_v)
            pltpu.async_copy(colflat_hbm.at[idx_v], buf_v, sem).wait()
            tgt = outs[c]
            wv = (None, w0_v, w1_v)[k]

            def chunk(i, _):
                sl = pl.ds(i * 16, 16)
                val = buf_v[sl]
                if wv is None:
                    tgt[sl] = val
                else:
                    tgt[sl] = tgt[sl] + wv[sl] * val
                return ()

            jax.lax.fori_loop(0, CS // 16, chunk, ())
        pltpu.sync_copy(ox_v, out_hbm.at[0, wid])
        pltpu.sync_copy(oy_v, out_hbm.at[1, wid])
        pltpu.sync_copy(oz_v, out_hbm.at[2, wid])

    return epi


def kernel(V, F):
    b = V.shape[0]
    V0 = V[:, F[:, 0]]
    V01 = V[:, F[:, 1]] - V0
    V02 = V[:, F[:, 2]] - V0
    pad = ((0, 0), (0, NFP - NF), (0, 0))
    v01p = jnp.pad(V01, pad).transpose(0, 2, 1)
    v02p = jnp.pad(V02, pad).transpose(0, 2, 1)
    logits = _face_logits(v01p, v02p)
    face_index = _face_sample_padded(logits)
    w0, w1 = _rand_scale_pair()
    w0 = w0.reshape(b, NSAMP)
    w1 = w1.reshape(b, NSAMP)
    flip = (w0 + w1) > 1.0
    w0 = jnp.abs(jnp.where(flip, w0 - 1.0, w0))
    w1 = jnp.abs(jnp.where(flip, w1 - 1.0, w1))
    colflat = jnp.stack([
        jnp.stack([Vk[:, :, c] for c in range(3)])
        for Vk in (V0, V01, V02)]).reshape(-1)
    gidx = face_index + (jnp.arange(b, dtype=jnp.int32) * NF)[:, None]
    gidx9 = (gidx[None] +
             (jnp.arange(9, dtype=jnp.int32) * (b * NF))[:, None, None])
    gidx9 = gidx9.reshape(9, 32, -1)
    out = _make_epilogue()(colflat, gidx9, w0.reshape(32, -1), w1.reshape(32, -1))
    return jnp.moveaxis(out.reshape(3, b, NSAMP), 0, -1)


def _face_sample_padded(lp):
    lp = lp.reshape(8, NCH, CH)
    nsteps = (8 * NSAMP) // (RB * GROUPS)
    steps_per_b = NSAMP // (RB * GROUPS)
    out = pl.pallas_call(
        functools.partial(_sample_body, nsamp=NSAMP),
        grid=(nsteps,),
        in_specs=[pl.BlockSpec((1, NCH, CH), lambda i: (i // steps_per_b, 0, 0))],
        out_specs=pl.BlockSpec((1, GROUPS, RB), lambda i: (i, 0, 0)),
        out_shape=jax.ShapeDtypeStruct((nsteps, GROUPS, RB), jnp.int32),
        compiler_params=pltpu.CompilerParams(
            dimension_semantics=("parallel",)),
    )(lp)
    return out.reshape(8, NSAMP)


# speedup vs baseline: 1.7444x; 1.0000x over previous
import functools

import jax
import jax.numpy as jnp
import numpy as np
from jax.experimental import pallas as pl
from jax.experimental.pallas import tpu as pltpu

NSAMP = 8192
NF = 100000
CH = 1024
NCH = 98
NFP = CH * NCH
RB = 8

_U32 = jnp.uint32


def _threefry(x0, x1, key1=42):
    k0 = np.uint32(0)
    k1 = np.uint32(key1)
    k2 = np.uint32(k0 ^ k1 ^ np.uint32(0x1BD11BDA))
    ks = [k0, k1, k2]
    rots = [[13, 15, 26, 6], [17, 29, 16, 24]]

    def rotl(x, d):
        return (x << _U32(d)) | (x >> _U32(32 - d))

    x1 = x1 + k1
    for r in range(5):
        for d in rots[r % 2]:
            x0 = x0 + x1
            x1 = rotl(x1, d)
            x1 = x0 ^ x1
        x0 = x0 + ks[(r + 1) % 3]
        x1 = x1 + np.uint32(ks[(r + 2) % 3] + np.uint32(r + 1))
    return x0, x1


GROUPS = 4


def _sample_body(l_ref, out_ref):
    pid = pl.program_id(0)
    for g in range(GROUPS):
        _sample_group(l_ref, out_ref, pid * (RB * GROUPS) + g * RB, g)


def _sample_group(l_ref, out_ref, r0, g):
    srow = r0 + jax.lax.broadcasted_iota(jnp.int32, (RB, 1), 0)
    s_u = srow.astype(_U32)
    base_lo = s_u * _U32(np.uint32(NF))
    full = srow.astype(jnp.float32) * np.float32(NF)
    base_hi = (full >= np.float32(2.0 ** 32)).astype(_U32)
    iota_u = jax.lax.broadcasted_iota(_U32, (1, CH), 1)

    def body(c, carry):
        maxval, maxidx = carry
        f0 = c * CH
        x1c = base_lo + f0.astype(_U32)
        x1 = x1c + iota_u
        hi = base_hi + (x1 < base_lo).astype(_U32)
        o0, o1 = _threefry(hi, x1)
        bits = o0 ^ o1
        fb = (bits >> _U32(9)) | _U32(0x3F800000)
        u = jax.lax.bitcast_convert_type(fb, jnp.float32) - np.float32(1.0)
        t = -jnp.log(-jnp.log(u)) + l_ref[0, c, :].reshape(1, CH)
        upd = t > maxval
        maxval = jnp.where(upd, t, maxval)
        maxidx = jnp.where(upd, x1, maxidx)
        return maxval, maxidx

    init = (jnp.full((RB, CH), -jnp.inf, jnp.float32),
            jnp.zeros((RB, CH), _U32))
    maxval, maxidx = jax.lax.fori_loop(0, NCH, body, init, unroll=28)
    fidx = (maxidx - base_lo).astype(jnp.int32)
    rm = jnp.max(maxval, axis=1, keepdims=True)
    cand = jnp.where(maxval == rm, fidx, jnp.int32(1 << 30))
    out_ref[0, g, :] = jnp.min(cand, axis=1)


def _w_body(o0_ref, o1_ref):
    b = pl.program_id(0)
    sv = jax.lax.broadcasted_iota(_U32, (1, NSAMP), 1)
    base = (_U32(b * NSAMP * 2) + sv * _U32(2))
    z = jnp.zeros_like(base)
    for j, ref in ((0, o0_ref), (1, o1_ref)):
        a0, a1 = _threefry(z, base + _U32(j), key1=43)
        bits = a0 ^ a1
        fb = (bits >> _U32(9)) | _U32(0x3F800000)
        ref[0] = jax.lax.bitcast_convert_type(fb, jnp.float32) - np.float32(1.0)


def _rand_scale_pair():
    return pl.pallas_call(
        _w_body,
        grid=(8,),
        in_specs=[],
        out_specs=[pl.BlockSpec((1, 1, NSAMP), lambda i: (i, 0, 0))] * 2,
        out_shape=[jax.ShapeDtypeStruct((8, 1, NSAMP), jnp.float32)] * 2,
    )()


def _geom_body(a_ref, b_ref, out_ref):
    a = a_ref[0]
    bb = b_ref[0]
    cx = a[1:2] * bb[2:3] - a[2:3] * bb[1:2]
    cy = a[2:3] * bb[0:1] - a[0:1] * bb[2:3]
    cz = a[0:1] * bb[1:2] - a[1:2] * bb[0:1]
    q = cx * cx + cy * cy + cz * cz
    area = np.float32(0.5) * jnp.sqrt(q)
    tot = jnp.sum(area, axis=1, keepdims=True)
    lg = jnp.log(area / tot + np.float32(1e-12))
    lane = jax.lax.broadcasted_iota(jnp.int32, (1, NFP), 1)
    out_ref[0] = jnp.where(lane < NF, lg, -jnp.inf)


def _face_logits(V01T, V02T):
    return pl.pallas_call(
        _geom_body,
        grid=(8,),
        in_specs=[pl.BlockSpec((1, 3, NFP), lambda i: (i, 0, 0))] * 2,
        out_specs=pl.BlockSpec((1, 1, NFP), lambda i: (i, 0, 0)),
        out_shape=jax.ShapeDtypeStruct((8, 1, NFP), jnp.float32),
    )(V01T, V02T).reshape(8, NFP)


def _make_epilogue():
    from jax import lax
    from jax.experimental.pallas import tpu_sc as plsc

    mesh = plsc.VectorSubcoreMesh(core_axis_name="c", subcore_axis_name="s")
    CS = (8 * NSAMP) // 32

    @functools.partial(
        pl.kernel, mesh=mesh,
        out_type=jax.ShapeDtypeStruct((3, 32, CS), jnp.float32),
        scratch_types=[
            pltpu.VMEM((CS,), jnp.int32),
            pltpu.VMEM((CS,), jnp.float32),
            pltpu.VMEM((CS,), jnp.float32),
            pltpu.VMEM((CS,), jnp.float32),
            pltpu.VMEM((CS,), jnp.float32),
            pltpu.VMEM((CS,), jnp.float32),
            pltpu.VMEM((CS,), jnp.float32),
            pltpu.SemaphoreType.DMA,
        ],
    )
    def epi(colflat_hbm, gidx9_hbm, w0_hbm, w1_hbm, out_hbm,
            idx_v, buf_v, w0_v, w1_v, ox_v, oy_v, oz_v, sem):
        wid = lax.axis_index("s") * 2 + lax.axis_index("c")
        pltpu.sync_copy(w0_hbm.at[wid], w0_v)
        pltpu.sync_copy(w1_hbm.at[wid], w1_v)
        outs = (ox_v, oy_v, oz_v)
        for t in range(9):
            k, c = divmod(t, 3)
            pltpu.sync_copy(gidx9_hbm.at[t, wid], idx_v)
            pltpu.async_copy(colflat_hbm.at[idx_v], buf_v, sem).wait()
            tgt = outs[c]
            wv = (None, w0_v, w1_v)[k]

            def chunk(i, _):
                sl = pl.ds(i * 16, 16)
                val = buf_v[sl]
                if wv is None:
                    tgt[sl] = val
                else:
                    tgt[sl] = tgt[sl] + wv[sl] * val
                return ()

            jax.lax.fori_loop(0, CS // 16, chunk, ())
        pltpu.sync_copy(ox_v, out_hbm.at[0, wid])
        pltpu.sync_copy(oy_v, out_hbm.at[1, wid])
        pltpu.sync_copy(oz_v, out_hbm.at[2, wid])

    return epi


def kernel(V, F):
    b = V.shape[0]
    V0 = V[:, F[:, 0]]
    V01 = V[:, F[:, 1]] - V0
    V02 = V[:, F[:, 2]] - V0
    pad = ((0, 0), (0, NFP - NF), (0, 0))
    v01p = jnp.pad(V01, pad).transpose(0, 2, 1)
    v02p = jnp.pad(V02, pad).transpose(0, 2, 1)
    logits = _face_logits(v01p, v02p)
    face_index = _face_sample_padded(logits)
    w0, w1 = _rand_scale_pair()
    w0 = w0.reshape(b, NSAMP)
    w1 = w1.reshape(b, NSAMP)
    flip = (w0 + w1) > 1.0
    w0 = jnp.abs(jnp.where(flip, w0 - 1.0, w0))
    w1 = jnp.abs(jnp.where(flip, w1 - 1.0, w1))
    colflat = jnp.stack([
        jnp.stack([Vk[:, :, c] for c in range(3)])
        for Vk in (V0, V01, V02)]).reshape(-1)
    gidx = face_index + (jnp.arange(b, dtype=jnp.int32) * NF)[:, None]
    gidx9 = (gidx[None] +
             (jnp.arange(9, dtype=jnp.int32) * (b * NF))[:, None, None])
    gidx9 = gidx9.reshape(9, 32, -1)
    out = _make_epilogue()(colflat, gidx9, w0.reshape(32, -1), w1.reshape(32, -1))
    return jnp.moveaxis(out.reshape(3, b, NSAMP), 0, -1)


def _face_sample_padded(lp):
    lp = lp.reshape(8, NCH, CH)
    nsteps = (8 * NSAMP) // (RB * GROUPS)
    steps_per_b = NSAMP // (RB * GROUPS)
    out = pl.pallas_call(
        _sample_body,
        grid=(nsteps,),
        in_specs=[pl.BlockSpec((1, NCH, CH), lambda i: (i // steps_per_b, 0, 0))],
        out_specs=pl.BlockSpec((1, GROUPS, RB), lambda i: (i, 0, 0)),
        out_shape=jax.ShapeDtypeStruct((nsteps, GROUPS, RB), jnp.int32),
        compiler_params=pltpu.CompilerParams(
            dimension_semantics=("parallel",)),
    )(lp)
    return out.reshape(8, NSAMP)
